# baseline (speedup 1.0000x reference)
import functools

import jax
import jax.numpy as jnp
from jax import lax
from jax.experimental import pallas as pl
from jax.experimental.pallas import tpu as pltpu
from jax.experimental.pallas import tpu_sc as plsc

RANK = 16
LANES = 16
NUM_WORKERS = 32
CHUNK = 992

_COMPILER_PARAMS = pltpu.CompilerParams(
    needs_layout_passes=False, use_tc_tiling_on_sc=False)


def _worker_range(wid, num_chunks):
    base_n = num_chunks // NUM_WORKERS
    rem = num_chunks % NUM_WORKERS
    extra = jnp.minimum(wid, rem)
    lo = wid * base_n + extra
    n = base_n + jnp.where(wid < rem, 1, 0)
    return lo, n


def _outer_iters(num_chunks):
    max_n = num_chunks // NUM_WORKERS + (1 if num_chunks % NUM_WORKERS else 0)
    return (max_n + 1) // 2


def _mul_kernel(nnz, ndim):
    num_chunks = -(-nnz // CHUNK)
    slab = ndim // 8

    mesh = plsc.VectorSubcoreMesh(core_axis_name="c", subcore_axis_name="s")

    @functools.partial(
        pl.kernel,
        mesh=mesh,
        compiler_params=_COMPILER_PARAMS,
        out_type=jax.ShapeDtypeStruct((nnz, RANK), jnp.float32),
        scratch_types=[
            pltpu.VMEM((CHUNK,), jnp.int32),
            pltpu.VMEM((CHUNK,), jnp.int32),
            pltpu.VMEM((CHUNK,), jnp.int32),
            pltpu.VMEM((CHUNK,), jnp.int32),
            pltpu.VMEM((CHUNK, RANK), jnp.float32),
            pltpu.VMEM((CHUNK, RANK), jnp.float32),
            pltpu.VMEM((CHUNK, RANK), jnp.float32),
            pltpu.VMEM((CHUNK, RANK), jnp.float32),
            pltpu.VMEM((CHUNK, RANK), jnp.float32),
            pltpu.VMEM((CHUNK, RANK), jnp.float32),
            pltpu.SemaphoreType.DMA,
            pltpu.SemaphoreType.DMA,
            pltpu.SemaphoreType.DMA,
            pltpu.SemaphoreType.DMA,
            pltpu.SemaphoreType.DMA,
            pltpu.SemaphoreType.DMA,
        ],
    )
    def k(i0_hbm, i1_hbm, f0_hbm, f1_hbm, p_hbm,
          i0a, i1a, i0b, i1b,
          r0a, r1a, r0b, r1b, pa, pb,
          sem_ia, sem_ib, sem_ga, sem_gb, sem_oa, sem_ob):
        wid = lax.axis_index("s") * 2 + lax.axis_index("c")
        lo, n = _worker_range(wid, num_chunks)
        idxs = ((i0a, i1a), (i0b, i1b))
        rows = ((r0a, r1a), (r0b, r1b))
        outs = (pa, pb)
        sems_i = (sem_ia, sem_ib)
        sems_g = (sem_ga, sem_gb)
        sems_o = (sem_oa, sem_ob)
        fs = (f0_hbm, f1_hbm)
        is_hbm = (i0_hbm, i1_hbm)

        def issue_idx(chunk_id, b):
            base = jnp.minimum(chunk_id * CHUNK, nnz - CHUNK)
            for m in range(2):
                pltpu.async_copy(is_hbm[m].at[pl.ds(base, CHUNK)],
                                 idxs[b][m], sems_i[b])

        def transform_idx(b):
            for m in range(2):
                ref = idxs[b][m]

                @plsc.parallel_loop(0, CHUNK // LANES)
                def tloop(g):
                    v = ref[pl.ds(g * LANES, LANES)]
                    q = v // slab
                    ref[pl.ds(g * LANES, LANES)] = (v - q * slab) * 8 + q

        def wait_idx(b):
            for m in range(2):
                pltpu.make_async_copy(is_hbm[m].at[pl.ds(0, CHUNK)],
                                      idxs[b][m], sems_i[b]).wait()

        def issue_gathers(b):
            for m in range(2):
                pltpu.async_copy(fs[m].at[idxs[b][m]], rows[b][m], sems_g[b])

        def wait_gathers(b):
            for m in range(2):
                pltpu.make_async_copy(fs[m].at[idxs[b][m]], rows[b][m],
                                      sems_g[b]).wait()

        def compute(b):
            r0, r1 = rows[b]
            p_v = outs[b]

            @plsc.parallel_loop(0, CHUNK)
            def row_body(c):
                p_v[c, :] = r0[c, :] * r1[c, :]

        def issue_out(kk, b):
            base = jnp.minimum((lo + kk) * CHUNK, nnz - CHUNK)
            pltpu.async_copy(outs[b], p_hbm.at[pl.ds(base, CHUNK), :],
                             sems_o[b])

        def wait_out(b):
            pltpu.make_async_copy(outs[b], p_hbm.at[pl.ds(0, CHUNK), :],
                                  sems_o[b]).wait()

        issue_idx(lo, 0)
        wait_idx(0)
        transform_idx(0)
        issue_gathers(0)
        issue_idx(lo + 1, 1)

        def body(kk, b):
            wait_gathers(b)

            @pl.when(kk + 1 < n)
            def _():
                wait_idx(1 - b)
                transform_idx(1 - b)
                issue_gathers(1 - b)

            @pl.when(kk + 2 < n)
            def _():
                issue_idx(lo + kk + 2, b)

            @pl.when(kk >= 2)
            def _():
                wait_out(b)

            compute(b)
            issue_out(kk, b)

        def outer(i, _):
            kk = i * 2

            @pl.when(kk < n)
            def _():
                body(kk, 0)

            @pl.when(kk + 1 < n)
            def _():
                body(kk + 1, 1)

            return 0

        lax.fori_loop(0, _outer_iters(num_chunks), outer, 0)
        wait_out(0)
        wait_out(1)

    return k


def _reduce_kernel(nnz, ndim):
    num_chunks = -(-nnz // CHUNK)
    slab = ndim // 8
    groups = CHUNK // LANES

    mesh = plsc.VectorSubcoreMesh(core_axis_name="c", subcore_axis_name="s")

    @functools.partial(
        pl.kernel,
        mesh=mesh,
        compiler_params=_COMPILER_PARAMS,
        out_type=jax.ShapeDtypeStruct((nnz,), jnp.float32),
        scratch_types=[
            pltpu.VMEM((CHUNK,), jnp.int32),
            pltpu.VMEM((CHUNK,), jnp.int32),
            pltpu.VMEM((CHUNK, RANK), jnp.float32),
            pltpu.VMEM((CHUNK, RANK), jnp.float32),
            pltpu.VMEM((CHUNK, RANK), jnp.float32),
            pltpu.VMEM((CHUNK, RANK), jnp.float32),
            pltpu.VMEM((CHUNK,), jnp.float32),
            pltpu.VMEM((CHUNK,), jnp.float32),
            pltpu.SemaphoreType.DMA,
            pltpu.SemaphoreType.DMA,
            pltpu.SemaphoreType.DMA,
            pltpu.SemaphoreType.DMA,
            pltpu.SemaphoreType.DMA,
            pltpu.SemaphoreType.DMA,
        ],
    )
    def k(i2_hbm, f2_hbm, p_hbm, out_hbm,
          i2a, i2b, r2a, r2b, pva, pvb, out_a, out_b,
          sem_ia, sem_ib, sem_ga, sem_gb, sem_oa, sem_ob):
        wid = lax.axis_index("s") * 2 + lax.axis_index("c")
        lo, n = _worker_range(wid, num_chunks)
        lane = lax.iota(jnp.int32, LANES)
        idxs = (i2a, i2b)
        rows = (r2a, r2b)
        pvs = (pva, pvb)
        outs = (out_a, out_b)
        sems_i = (sem_ia, sem_ib)
        sems_g = (sem_ga, sem_gb)
        sems_o = (sem_oa, sem_ob)

        def issue_idx(chunk_id, b):
            base = jnp.minimum(chunk_id * CHUNK, nnz - CHUNK)
            pltpu.async_copy(i2_hbm.at[pl.ds(base, CHUNK)], idxs[b],
                             sems_i[b])

        def transform_idx(b):
            ref = idxs[b]

            @plsc.parallel_loop(0, CHUNK // LANES)
            def tloop(g):
                v = ref[pl.ds(g * LANES, LANES)]
                q = v // slab
                ref[pl.ds(g * LANES, LANES)] = (v - q * slab) * 8 + q

        def wait_idx(b):
            pltpu.make_async_copy(i2_hbm.at[pl.ds(0, CHUNK)], idxs[b],
                                  sems_i[b]).wait()

        def issue_gathers(chunk_id, b):
            base = jnp.minimum(chunk_id * CHUNK, nnz - CHUNK)
            pltpu.async_copy(f2_hbm.at[idxs[b]], rows[b], sems_g[b])
            pltpu.async_copy(p_hbm.at[pl.ds(base, CHUNK), :], pvs[b],
                             sems_g[b])

        def wait_gathers(b):
            pltpu.make_async_copy(f2_hbm.at[idxs[b]], rows[b],
                                  sems_g[b]).wait()
            pltpu.make_async_copy(p_hbm.at[pl.ds(0, CHUNK), :], pvs[b],
                                  sems_g[b]).wait()

        def compute(b):
            r2, p_v, out_v = rows[b], pvs[b], outs[b]

            @plsc.parallel_loop(0, groups)
            def group_body(g):
                off = g * LANES
                row_ids = off + lane
                acc = jnp.zeros((LANES,), jnp.float32)
                for r in range(RANK):
                    col = jnp.full((LANES,), r, jnp.int32)
                    vp = plsc.load_gather(p_v, [row_ids, col])
                    v2 = plsc.load_gather(r2, [row_ids, col])
                    acc = acc + vp * v2
                out_v[pl.ds(off, LANES)] = acc

        def issue_out(kk, b):
            base = jnp.minimum((lo + kk) * CHUNK, nnz - CHUNK)
            pltpu.async_copy(outs[b], out_hbm.at[pl.ds(base, CHUNK)],
                             sems_o[b])

        def wait_out(b):
            pltpu.make_async_copy(outs[b], out_hbm.at[pl.ds(0, CHUNK)],
                                  sems_o[b]).wait()

        issue_idx(lo, 0)
        wait_idx(0)
        transform_idx(0)
        issue_gathers(lo, 0)
        issue_idx(lo + 1, 1)

        def body(kk, b):
            wait_gathers(b)

            @pl.when(kk + 1 < n)
            def _():
                wait_idx(1 - b)
                transform_idx(1 - b)
                issue_gathers(lo + kk + 1, 1 - b)

            @pl.when(kk + 2 < n)
            def _():
                issue_idx(lo + kk + 2, b)

            @pl.when(kk >= 2)
            def _():
                wait_out(b)

            compute(b)
            issue_out(kk, b)

        def outer(i, _):
            kk = i * 2

            @pl.when(kk < n)
            def _():
                body(kk, 0)

            @pl.when(kk + 1 < n)
            def _():
                body(kk + 1, 1)

            return 0

        lax.fori_loop(0, _outer_iters(num_chunks), outer, 0)
        wait_out(0)
        wait_out(1)

    return k


RELAYOUT_ROWS = 1000


def _relayout_kernel(ndim):
    slab = ndim // 8
    nblk = slab // RELAYOUT_ROWS
    assert nblk * RELAYOUT_ROWS == slab

    def body(*refs):
        xs = refs[:8]
        o_ref = refs[8]
        o_ref[...] = jnp.concatenate([x[...] for x in xs], axis=1)

    def make_spec(s):
        return pl.BlockSpec((RELAYOUT_ROWS, RANK),
                            lambda b, s=s: (s * nblk + b, 0))

    f = pl.pallas_call(
        body,
        grid=(nblk,),
        in_specs=[make_spec(s) for s in range(8)],
        out_specs=pl.BlockSpec((RELAYOUT_ROWS, 8 * RANK), lambda b: (b, 0)),
        out_shape=jax.ShapeDtypeStruct((slab, 8 * RANK), jnp.float32),
    )
    return lambda x: f(x, x, x, x, x, x, x, x)


def kernel(indices_list, f0, f1, f2):
    nnz = indices_list.shape[1]
    ndim = f0.shape[0]
    idx = indices_list.astype(jnp.int32)
    relayout = _relayout_kernel(ndim)
    g0 = relayout(f0).reshape(ndim, RANK)
    g1 = relayout(f1).reshape(ndim, RANK)
    g2 = relayout(f2).reshape(ndim, RANK)
    p = _mul_kernel(nnz, ndim)(idx[0], idx[1], g0, g1)
    return _reduce_kernel(nnz, ndim)(idx[2], g2, p)


# speedup vs baseline: 1.3176x; 1.3176x over previous
import functools

import jax
import jax.numpy as jnp
from jax import lax
from jax.experimental import pallas as pl
from jax.experimental.pallas import tpu as pltpu
from jax.experimental.pallas import tpu_sc as plsc

RANK = 16
LANES = 16
NUM_WORKERS = 32
CHUNK_A = 992
CHUNK_B = 1600

_COMPILER_PARAMS = pltpu.CompilerParams(
    needs_layout_passes=False, use_tc_tiling_on_sc=False)


def _worker_range(wid, num_chunks):
    base_n = num_chunks // NUM_WORKERS
    rem = num_chunks % NUM_WORKERS
    extra = jnp.minimum(wid, rem)
    lo = wid * base_n + extra
    n = base_n + jnp.where(wid < rem, 1, 0)
    return lo, n


def _outer_iters(num_chunks):
    max_n = num_chunks // NUM_WORKERS + (1 if num_chunks % NUM_WORKERS else 0)
    return (max_n + 1) // 2


def _mul_kernel(nnz, ndim):
    chunk = CHUNK_A
    num_chunks = -(-nnz // chunk)

    mesh = plsc.VectorSubcoreMesh(core_axis_name="c", subcore_axis_name="s")

    @functools.partial(
        pl.kernel,
        mesh=mesh,
        compiler_params=_COMPILER_PARAMS,
        out_type=jax.ShapeDtypeStruct((nnz, RANK), jnp.float32),
        scratch_types=[
            pltpu.VMEM((chunk,), jnp.int32),
            pltpu.VMEM((chunk,), jnp.int32),
            pltpu.VMEM((chunk,), jnp.int32),
            pltpu.VMEM((chunk,), jnp.int32),
            pltpu.VMEM((chunk, RANK), jnp.float32),
            pltpu.VMEM((chunk, RANK), jnp.float32),
            pltpu.VMEM((chunk, RANK), jnp.float32),
            pltpu.VMEM((chunk, RANK), jnp.float32),
            pltpu.VMEM((chunk, RANK), jnp.float32),
            pltpu.VMEM((chunk, RANK), jnp.float32),
            pltpu.SemaphoreType.DMA,
            pltpu.SemaphoreType.DMA,
            pltpu.SemaphoreType.DMA,
            pltpu.SemaphoreType.DMA,
            pltpu.SemaphoreType.DMA,
            pltpu.SemaphoreType.DMA,
        ],
    )
    def k(i0_hbm, i1_hbm, f0_hbm, f1_hbm, p_hbm,
          i0a, i1a, i0b, i1b,
          r0a, r1a, r0b, r1b, pa, pb,
          sem_ia, sem_ib, sem_ga, sem_gb, sem_oa, sem_ob):
        wid = lax.axis_index("s") * 2 + lax.axis_index("c")
        lo, n = _worker_range(wid, num_chunks)
        idxs = ((i0a, i1a), (i0b, i1b))
        rows = ((r0a, r1a), (r0b, r1b))
        outs = (pa, pb)
        sems_i = (sem_ia, sem_ib)
        sems_g = (sem_ga, sem_gb)
        sems_o = (sem_oa, sem_ob)
        fs = (f0_hbm, f1_hbm)
        is_hbm = (i0_hbm, i1_hbm)

        def issue_idx(chunk_id, b):
            base = jnp.minimum(chunk_id * chunk, nnz - chunk)
            for m in range(2):
                pltpu.async_copy(is_hbm[m].at[pl.ds(base, chunk)],
                                 idxs[b][m], sems_i[b])

        def wait_idx(b):
            for m in range(2):
                pltpu.make_async_copy(is_hbm[m].at[pl.ds(0, chunk)],
                                      idxs[b][m], sems_i[b]).wait()

        def issue_gathers(b):
            for m in range(2):
                pltpu.async_copy(fs[m].at[idxs[b][m]], rows[b][m], sems_g[b])

        def wait_gathers(b):
            for m in range(2):
                pltpu.make_async_copy(fs[m].at[idxs[b][m]], rows[b][m],
                                      sems_g[b]).wait()

        def compute(b):
            r0, r1 = rows[b]
            p_v = outs[b]

            @plsc.parallel_loop(0, chunk)
            def row_body(c):
                p_v[c, :] = r0[c, :] * r1[c, :]

        def issue_out(kk, b):
            base = jnp.minimum((lo + kk) * chunk, nnz - chunk)
            pltpu.async_copy(outs[b], p_hbm.at[pl.ds(base, chunk), :],
                             sems_o[b])

        def wait_out(b):
            pltpu.make_async_copy(outs[b], p_hbm.at[pl.ds(0, chunk), :],
                                  sems_o[b]).wait()

        issue_idx(lo, 0)
        wait_idx(0)
        issue_gathers(0)
        issue_idx(lo + 1, 1)

        def body(kk, b):
            wait_gathers(b)

            @pl.when(kk + 1 < n)
            def _():
                wait_idx(1 - b)
                issue_gathers(1 - b)

            @pl.when(kk + 2 < n)
            def _():
                issue_idx(lo + kk + 2, b)

            @pl.when(kk >= 2)
            def _():
                wait_out(b)

            compute(b)
            issue_out(kk, b)

        def outer(i, _):
            kk = i * 2

            @pl.when(kk < n)
            def _():
                body(kk, 0)

            @pl.when(kk + 1 < n)
            def _():
                body(kk + 1, 1)

            return 0

        lax.fori_loop(0, _outer_iters(num_chunks), outer, 0)
        wait_out(0)
        wait_out(1)

    return k


def _reduce_kernel(nnz, ndim):
    chunk = CHUNK_B
    num_chunks = -(-nnz // chunk)
    groups = chunk // LANES

    mesh = plsc.VectorSubcoreMesh(core_axis_name="c", subcore_axis_name="s")

    @functools.partial(
        pl.kernel,
        mesh=mesh,
        compiler_params=_COMPILER_PARAMS,
        out_type=jax.ShapeDtypeStruct((nnz,), jnp.float32),
        scratch_types=[
            pltpu.VMEM((chunk,), jnp.int32),
            pltpu.VMEM((chunk,), jnp.int32),
            pltpu.VMEM((chunk, RANK), jnp.float32),
            pltpu.VMEM((chunk, RANK), jnp.float32),
            pltpu.VMEM((chunk, RANK), jnp.float32),
            pltpu.VMEM((chunk, RANK), jnp.float32),
            pltpu.VMEM((chunk,), jnp.float32),
            pltpu.VMEM((chunk,), jnp.float32),
            pltpu.SemaphoreType.DMA,
            pltpu.SemaphoreType.DMA,
            pltpu.SemaphoreType.DMA,
            pltpu.SemaphoreType.DMA,
            pltpu.SemaphoreType.DMA,
            pltpu.SemaphoreType.DMA,
        ],
    )
    def k(i2_hbm, f2_hbm, p_hbm, out_hbm,
          i2a, i2b, r2a, r2b, pva, pvb, out_a, out_b,
          sem_ia, sem_ib, sem_ga, sem_gb, sem_oa, sem_ob):
        wid = lax.axis_index("s") * 2 + lax.axis_index("c")
        lo, n = _worker_range(wid, num_chunks)
        lane = lax.iota(jnp.int32, LANES)
        idxs = (i2a, i2b)
        rows = (r2a, r2b)
        pvs = (pva, pvb)
        outs = (out_a, out_b)
        sems_i = (sem_ia, sem_ib)
        sems_g = (sem_ga, sem_gb)
        sems_o = (sem_oa, sem_ob)

        def issue_idx(chunk_id, b):
            base = jnp.minimum(chunk_id * chunk, nnz - chunk)
            pltpu.async_copy(i2_hbm.at[pl.ds(base, chunk)], idxs[b],
                             sems_i[b])

        def wait_idx(b):
            pltpu.make_async_copy(i2_hbm.at[pl.ds(0, chunk)], idxs[b],
                                  sems_i[b]).wait()

        def issue_gathers(chunk_id, b):
            base = jnp.minimum(chunk_id * chunk, nnz - chunk)
            pltpu.async_copy(f2_hbm.at[idxs[b]], rows[b], sems_g[b])
            pltpu.async_copy(p_hbm.at[pl.ds(base, chunk), :], pvs[b],
                             sems_g[b])

        def wait_gathers(b):
            pltpu.make_async_copy(f2_hbm.at[idxs[b]], rows[b],
                                  sems_g[b]).wait()
            pltpu.make_async_copy(p_hbm.at[pl.ds(0, chunk), :], pvs[b],
                                  sems_g[b]).wait()

        def compute(b):
            r2, p_v, out_v = rows[b], pvs[b], outs[b]

            @plsc.parallel_loop(0, groups)
            def group_body(g):
                off = g * LANES
                row_ids = off + lane
                acc = jnp.zeros((LANES,), jnp.float32)
                for r in range(RANK):
                    col = jnp.full((LANES,), r, jnp.int32)
                    vp = plsc.load_gather(p_v, [row_ids, col])
                    v2 = plsc.load_gather(r2, [row_ids, col])
                    acc = acc + vp * v2
                out_v[pl.ds(off, LANES)] = acc

        def issue_out(kk, b):
            base = jnp.minimum((lo + kk) * chunk, nnz - chunk)
            pltpu.async_copy(outs[b], out_hbm.at[pl.ds(base, chunk)],
                             sems_o[b])

        def wait_out(b):
            pltpu.make_async_copy(outs[b], out_hbm.at[pl.ds(0, chunk)],
                                  sems_o[b]).wait()

        issue_idx(lo, 0)
        wait_idx(0)
        issue_gathers(lo, 0)
        issue_idx(lo + 1, 1)

        def body(kk, b):
            wait_gathers(b)

            @pl.when(kk + 1 < n)
            def _():
                wait_idx(1 - b)
                issue_gathers(lo + kk + 1, 1 - b)

            @pl.when(kk + 2 < n)
            def _():
                issue_idx(lo + kk + 2, b)

            @pl.when(kk >= 2)
            def _():
                wait_out(b)

            compute(b)
            issue_out(kk, b)

        def outer(i, _):
            kk = i * 2

            @pl.when(kk < n)
            def _():
                body(kk, 0)

            @pl.when(kk + 1 < n)
            def _():
                body(kk + 1, 1)

            return 0

        lax.fori_loop(0, _outer_iters(num_chunks), outer, 0)
        wait_out(0)
        wait_out(1)

    return k


def kernel(indices_list, f0, f1, f2):
    nnz = indices_list.shape[1]
    ndim = f0.shape[0]
    idx = indices_list.astype(jnp.int32)
    p = _mul_kernel(nnz, ndim)(idx[0], idx[1], f0, f1)
    return _reduce_kernel(nnz, ndim)(idx[2], f2, p)
